# Initial kernel scaffold; baseline (speedup 1.0000x reference)
import jax
import jax.numpy as jnp
from jax.experimental import pallas as pl


def kernel(emb_user, feats_item, W_item, nid_user, nid_item):
    raise NotImplementedError("write your pallas kernel here")


# speedup vs baseline: 1.0915x; 1.0915x over previous
import functools

import jax
import jax.numpy as jnp
from jax import lax
from jax.experimental import pallas as pl
from jax.experimental.pallas import tpu as pltpu
from jax.experimental.pallas import tpu_sc as plsc

B = 50000
EMB = 64
DFEAT = 128

_INFO = plsc.get_sparse_core_info()
_NC = _INFO.num_cores
_NS = _INFO.num_subcores
_NW = _NC * _NS

CHUNK = 80
_NCHUNKS = B // CHUNK
_MAXC = -(-_NCHUNKS // _NW)
_FULL = _NCHUNKS - (_MAXC - 1) * _NW
_SLAB = _MAXC * CHUNK
_NID_PAD = (_NW - 1) * (_MAXC - 1) * CHUNK + _FULL * CHUNK + _SLAB - B


def _sc_body(emb_hbm, feats_hbm, nidu_hbm, nidi_hbm,
             outu_hbm, rows_hbm,
             idxu_v, idxi_v, bufu_v, bufi_v, semu, semi):
    wid = lax.axis_index("s") * _NC + lax.axis_index("c")
    nchunks = jnp.where(wid < _FULL, _MAXC, _MAXC - 1)
    base = wid * ((_MAXC - 1) * CHUNK) + jnp.minimum(wid, _FULL) * CHUNK

    pltpu.sync_copy(nidu_hbm.at[pl.ds(base, _SLAB)], idxu_v)
    pltpu.sync_copy(nidi_hbm.at[pl.ds(base, _SLAB)], idxi_v)

    for j in range(_MAXC):
        @pl.when(j < nchunks)
        def _():
            off = j * CHUNK
            cu = pltpu.async_copy(
                emb_hbm.at[idxu_v.at[pl.ds(off, CHUNK)]], bufu_v, semu)
            ci = pltpu.async_copy(
                feats_hbm.at[idxi_v.at[pl.ds(off, CHUNK)]], bufi_v, semi)
            cu.wait()
            pltpu.sync_copy(bufu_v, outu_hbm.at[pl.ds(base + off, CHUNK)])
            ci.wait()
            pltpu.sync_copy(bufi_v, rows_hbm.at[pl.ds(base + off, CHUNK)])


_sc_gather = functools.partial(
    pl.kernel,
    mesh=plsc.VectorSubcoreMesh(core_axis_name="c", subcore_axis_name="s"),
    out_type=[
        jax.ShapeDtypeStruct((B, EMB), jnp.float32),
        jax.ShapeDtypeStruct((B, DFEAT), jnp.float32),
    ],
    scratch_types=[
        pltpu.VMEM((_SLAB,), jnp.int32),
        pltpu.VMEM((_SLAB,), jnp.int32),
        pltpu.VMEM((CHUNK, EMB), jnp.float32),
        pltpu.VMEM((CHUNK, DFEAT), jnp.float32),
        pltpu.SemaphoreType.DMA,
        pltpu.SemaphoreType.DMA,
    ],
    compiler_params=pltpu.CompilerParams(use_tc_tiling_on_sc=False),
)(_sc_body)


def _mm_body(x_ref, w_ref, o_ref):
    o_ref[...] = jnp.dot(x_ref[...], w_ref[...],
                         preferred_element_type=jnp.float32)


_MM_BLK = 2000


def _project(rows, w):
    return pl.pallas_call(
        _mm_body,
        grid=(B // _MM_BLK,),
        in_specs=[
            pl.BlockSpec((_MM_BLK, DFEAT), lambda i: (i, 0)),
            pl.BlockSpec((DFEAT, EMB), lambda i: (0, 0)),
        ],
        out_specs=pl.BlockSpec((_MM_BLK, EMB), lambda i: (i, 0)),
        out_shape=jax.ShapeDtypeStruct((B, EMB), jnp.float32),
    )(rows, w)


def kernel(emb_user, feats_item, W_item, nid_user, nid_item):
    nid_u = jnp.pad(nid_user.astype(jnp.int32), (0, _NID_PAD))
    nid_i = jnp.pad(nid_item.astype(jnp.int32), (0, _NID_PAD))
    x_user, rows = _sc_gather(emb_user, feats_item, nid_u, nid_i)
    x_item = _project(rows, W_item)
    return (x_user, x_item)
